# baseline (speedup 1.0000x reference)
import functools

import jax
import jax.numpy as jnp
from jax.experimental import pallas as pl
from jax.experimental.pallas import tpu as pltpu


def _mean_tree_kernel(x_ref, o_ref, *, num_neigh, inv_n):
    del num_neigh
    s = jnp.sum(x_ref[...].astype(jnp.float32), axis=1)
    o_ref[...] = (s * inv_n).astype(o_ref.dtype)


def _pick_tile(num_src, row_bytes):
    target = 4 << 20
    tile = max(8, min(num_src, target // max(row_bytes, 1)))
    tile -= tile % 8
    t = tile
    while t >= 8:
        if num_src % t == 0:
            return t
        t -= 8
    return max(tile, 8)


def kernel(neighbor_feature):
    num_src, num_neigh, input_dim = neighbor_feature.shape
    dtype = neighbor_feature.dtype
    itemsize = jnp.dtype(dtype).itemsize

    row_bytes = num_neigh * input_dim * itemsize
    tile = _pick_tile(num_src, row_bytes)
    grid = (pl.cdiv(num_src, tile),)

    kfn = functools.partial(
        _mean_tree_kernel, num_neigh=num_neigh, inv_n=1.0 / float(num_neigh))

    in_bytes = tile * row_bytes
    out_bytes = tile * input_dim * itemsize
    vmem_limit = int(min(100 << 20, 2 * in_bytes + 2 * out_bytes + (4 << 20)))

    return pl.pallas_call(
        kfn,
        out_shape=jax.ShapeDtypeStruct((num_src, input_dim), dtype),
        grid=grid,
        in_specs=[pl.BlockSpec((tile, num_neigh, input_dim),
                               lambda i: (i, 0, 0))],
        out_specs=pl.BlockSpec((tile, input_dim), lambda i: (i, 0)),
        compiler_params=pltpu.CompilerParams(
            dimension_semantics=("parallel",),
            vmem_limit_bytes=vmem_limit,
        ),
        cost_estimate=pl.CostEstimate(
            flops=num_src * num_neigh * input_dim,
            transcendentals=0,
            bytes_accessed=num_src * (num_neigh + 1) * input_dim * itemsize,
        ),
    )(neighbor_feature)


# speedup vs baseline: 1.1406x; 1.1406x over previous
import functools

import jax
import jax.numpy as jnp
from jax.experimental import pallas as pl
from jax.experimental.pallas import tpu as pltpu


def _mean_tree_kernel(x_ref, o_ref, *, num_neigh, inv_n):
    del num_neigh
    s = jnp.sum(x_ref[...].astype(jnp.float32), axis=1)
    o_ref[...] = (s * inv_n).astype(o_ref.dtype)


def _pick_tile(num_src, row_bytes):
    target = 16 << 20
    tile = max(8, min(num_src, target // max(row_bytes, 1)))
    tile -= tile % 8
    t = tile
    while t >= 8:
        if num_src % t == 0:
            return t
        t -= 8
    return max(tile, 8)


def kernel(neighbor_feature):
    num_src, num_neigh, input_dim = neighbor_feature.shape
    dtype = neighbor_feature.dtype
    itemsize = jnp.dtype(dtype).itemsize

    row_bytes = num_neigh * input_dim * itemsize
    tile = _pick_tile(num_src, row_bytes)
    grid = (pl.cdiv(num_src, tile),)

    kfn = functools.partial(
        _mean_tree_kernel, num_neigh=num_neigh, inv_n=1.0 / float(num_neigh))

    in_bytes = tile * row_bytes
    out_bytes = tile * input_dim * itemsize
    vmem_limit = int(min(100 << 20, 2 * in_bytes + 2 * out_bytes + (4 << 20)))

    return pl.pallas_call(
        kfn,
        out_shape=jax.ShapeDtypeStruct((num_src, input_dim), dtype),
        grid=grid,
        in_specs=[pl.BlockSpec((tile, num_neigh, input_dim),
                               lambda i: (i, 0, 0))],
        out_specs=pl.BlockSpec((tile, input_dim), lambda i: (i, 0)),
        compiler_params=pltpu.CompilerParams(
            dimension_semantics=("parallel",),
            vmem_limit_bytes=vmem_limit,
        ),
        cost_estimate=pl.CostEstimate(
            flops=num_src * num_neigh * input_dim,
            transcendentals=0,
            bytes_accessed=num_src * (num_neigh + 1) * input_dim * itemsize,
        ),
    )(neighbor_feature)


# speedup vs baseline: 1.1458x; 1.0046x over previous
import functools

import jax
import jax.numpy as jnp
from jax.experimental import pallas as pl
from jax.experimental.pallas import tpu as pltpu


def _mean_kernel(x_ref, o_ref, *, tile, inv_n):
    j = pl.program_id(1)
    s = jnp.sum(x_ref[...].astype(jnp.float32), axis=1)
    o_ref[pl.ds(j * tile, tile), :] = (s * inv_n).astype(o_ref.dtype)


def kernel(neighbor_feature):
    num_src, num_neigh, input_dim = neighbor_feature.shape
    dtype = neighbor_feature.dtype
    itemsize = jnp.dtype(dtype).itemsize

    row_bytes = num_neigh * input_dim * itemsize

    tile = max(8, min(num_src, (8 << 20) // max(row_bytes, 1)))
    tile -= tile % 8
    while tile > 8 and num_src % tile != 0:
        tile -= 8

    n_blocks = num_src // tile
    if n_blocks % 2 == 0 and n_blocks >= 2:
        n_cores, steps = 2, n_blocks // 2
    else:
        n_cores, steps = 1, n_blocks
    rows_per_core = num_src // n_cores

    kfn = functools.partial(_mean_kernel, tile=tile,
                            inv_n=1.0 / float(num_neigh))

    in_bytes = tile * row_bytes
    out_bytes = rows_per_core * input_dim * itemsize
    vmem_limit = int(min(100 << 20, 2 * in_bytes + 2 * out_bytes + (8 << 20)))

    return pl.pallas_call(
        kfn,
        out_shape=jax.ShapeDtypeStruct((num_src, input_dim), dtype),
        grid=(n_cores, steps),
        in_specs=[pl.BlockSpec(
            (tile, num_neigh, input_dim),
            lambda i, j, s=steps: (i * s + j, 0, 0))],
        out_specs=pl.BlockSpec((rows_per_core, input_dim), lambda i, j: (i, 0)),
        compiler_params=pltpu.CompilerParams(
            dimension_semantics=("parallel", "arbitrary"),
            vmem_limit_bytes=vmem_limit,
        ),
        cost_estimate=pl.CostEstimate(
            flops=num_src * num_neigh * input_dim,
            transcendentals=0,
            bytes_accessed=num_src * (num_neigh + 1) * input_dim * itemsize,
        ),
    )(neighbor_feature)


# speedup vs baseline: 1.1547x; 1.0077x over previous
import functools

import jax
import jax.numpy as jnp
from jax.experimental import pallas as pl
from jax.experimental.pallas import tpu as pltpu


def _mean_kernel(x_ref, o_ref, *, num_neigh, inv_n):
    x = x_ref[...].astype(jnp.float32)
    half = num_neigh // 2
    if num_neigh % 2 == 0 and half % 8 == 0:
        s = jnp.sum(x[:, :half, :] + x[:, half:, :], axis=1)
    else:
        s = jnp.sum(x, axis=1)
    o_ref[...] = (s * inv_n).astype(o_ref.dtype)


def _pick_tile(num_src, row_bytes):
    tile = max(8, min(num_src, (8 << 20) // max(row_bytes, 1)))
    tile -= tile % 8
    while tile > 8 and num_src % tile != 0:
        tile -= 8
    return tile


def kernel(neighbor_feature):
    num_src, num_neigh, input_dim = neighbor_feature.shape
    dtype = neighbor_feature.dtype
    itemsize = jnp.dtype(dtype).itemsize

    row_bytes = num_neigh * input_dim * itemsize
    tile = _pick_tile(num_src, row_bytes)
    grid = (pl.cdiv(num_src, tile),)

    kfn = functools.partial(_mean_kernel, num_neigh=num_neigh,
                            inv_n=1.0 / float(num_neigh))

    in_bytes = tile * row_bytes
    out_bytes = tile * input_dim * itemsize
    vmem_limit = int(min(100 << 20, 2 * in_bytes + 2 * out_bytes + (4 << 20)))

    return pl.pallas_call(
        kfn,
        out_shape=jax.ShapeDtypeStruct((num_src, input_dim), dtype),
        grid=grid,
        in_specs=[pl.BlockSpec((tile, num_neigh, input_dim),
                               lambda i: (i, 0, 0))],
        out_specs=pl.BlockSpec((tile, input_dim), lambda i: (i, 0)),
        compiler_params=pltpu.CompilerParams(
            dimension_semantics=("parallel",),
            vmem_limit_bytes=vmem_limit,
        ),
        cost_estimate=pl.CostEstimate(
            flops=num_src * num_neigh * input_dim,
            transcendentals=0,
            bytes_accessed=num_src * (num_neigh + 1) * input_dim * itemsize,
        ),
    )(neighbor_feature)
